# baseline (speedup 1.0000x reference)
import functools

import jax
import jax.numpy as jnp
import numpy as np
from jax import lax
from jax.experimental import pallas as pl
from jax.experimental.pallas import tpu as pltpu
from jax.experimental.pallas import tpu_sc as plsc

_BLOCK = 64
_TOPK = 8
_LANES = 16


def _scores_body(q_ref, k_ref, s_ref, *, M, L, D, G):
    for g in range(G):
        qp = jnp.mean(q_ref[g].reshape(M, _BLOCK, D), axis=1)
        kp = jnp.mean(k_ref[g].reshape(M, _BLOCK, D), axis=1)
        s_ref[g] = jax.lax.dot_general(
            qp.astype(jnp.bfloat16), kp.astype(jnp.bfloat16),
            (((1,), (1,)), ((), ())),
            preferred_element_type=jnp.float32)


def _sc_topk_body(scores_hbm, lut_hbm, sv_ref, lv_ref, *, rows_per_worker,
                  num_cores):
    wid = lax.axis_index("s") * num_cores + lax.axis_index("c")
    base = wid * rows_per_worker
    pltpu.sync_copy(scores_hbm.at[pl.ds(base, rows_per_worker)], sv_ref)
    ii = lax.iota(jnp.int32, _LANES)
    neg_inf = jnp.float32(-jnp.inf)

    gdims = lax.GatherDimensionNumbers(
        offset_dims=(), collapsed_slice_dims=(0,), start_index_map=(0,))

    def _rot(x, sh):
        perm = (ii + sh) & (_LANES - 1)
        return lax.gather(x, perm[:, None], gdims, slice_sizes=(1,),
                          mode=lax.GatherScatterMode.PROMISE_IN_BOUNDS)

    def vmax_splat(x):
        for sh in (8, 4, 2, 1):
            x = jnp.maximum(x, _rot(x, sh))
        return x

    def vmin_splat(x):
        for sh in (8, 4, 2, 1):
            x = jnp.minimum(x, _rot(x, sh))
        return x

    def first_idx_of(x, mx):
        return vmin_splat(jnp.where(x == mx, ii, _LANES))

    for r in range(rows_per_worker):
        a = sv_ref[r, pl.ds(0, _LANES)]
        b = sv_ref[r, pl.ds(_LANES, _LANES)]
        lutrow = jnp.zeros((_LANES,), jnp.int32)
        for t in range(_TOPK):
            ma = vmax_splat(a)
            mb = vmax_splat(b)
            fa = first_idx_of(a, ma)
            fb = first_idx_of(b, mb)
            idxv = jnp.where(ma >= mb, fa, fb + _LANES)
            a = jnp.where(ii == idxv, neg_inf, a)
            b = jnp.where(ii + _LANES == idxv, neg_inf, b)
            lutrow = jnp.where(ii == t, idxv, lutrow)
        lv_ref[r, :] = lutrow
    pltpu.sync_copy(lv_ref, lut_hbm.at[pl.ds(base, rows_per_worker)])


def _attn_body(lut_ref, q_ref, k_ref, v_ref, o_ref, kb_ref, vb_ref,
               s_ref, p_ref, *, M, scale):
    bh = pl.program_id(0)
    kb_ref[...] = k_ref[0].astype(jnp.bfloat16)
    vb_ref[...] = v_ref[0].astype(jnp.bfloat16)
    for m in range(M):
        base = (bh * M + m) * _LANES
        q = q_ref[0, m * _BLOCK:(m + 1) * _BLOCK, :].astype(jnp.bfloat16)
        kcat = jnp.concatenate(
            [kb_ref[pl.ds(lut_ref[base + t] * _BLOCK, _BLOCK), :]
             for t in range(_TOPK)], axis=0)
        s_ref[m * _BLOCK:(m + 1) * _BLOCK, :] = jax.lax.dot_general(
            q, kcat, (((1,), (1,)), ((), ())),
            preferred_element_type=jnp.float32) * scale
    for m in range(M):
        s = s_ref[m * _BLOCK:(m + 1) * _BLOCK, :]
        mx = jnp.max(s, axis=-1, keepdims=True)
        p = jnp.exp(s - mx)
        se = jnp.sum(p, axis=-1, keepdims=True)
        p_ref[m * _BLOCK:(m + 1) * _BLOCK, :] = (p / se).astype(jnp.bfloat16)
    for m in range(M):
        base = (bh * M + m) * _LANES
        vcat = jnp.concatenate(
            [vb_ref[pl.ds(lut_ref[base + t] * _BLOCK, _BLOCK), :]
             for t in range(_TOPK)], axis=0)
        o_ref[0, m * _BLOCK:(m + 1) * _BLOCK, :] = jax.lax.dot_general(
            p_ref[m * _BLOCK:(m + 1) * _BLOCK, :], vcat,
            (((1,), (0,)), ((), ())), preferred_element_type=jnp.float32)


def kernel(q, k, v):
    B, H, L, D = q.shape
    M = L // _BLOCK
    BH = B * H
    q3 = q.reshape(BH, L, D)
    k3 = k.reshape(BH, L, D)
    v3 = v.reshape(BH, L, D)

    G = 4
    scores = pl.pallas_call(
        functools.partial(_scores_body, M=M, L=L, D=D, G=G),
        grid=(BH // G,),
        in_specs=[pl.BlockSpec((G, L, D), lambda i: (i, 0, 0)),
                  pl.BlockSpec((G, L, D), lambda i: (i, 0, 0))],
        out_specs=pl.BlockSpec((G, M, M), lambda i: (i, 0, 0)),
        out_shape=jax.ShapeDtypeStruct((BH, M, M), jnp.float32),
    )(q3, k3)

    info = plsc.get_sparse_core_info()
    num_cores, num_subcores = info.num_cores, info.num_subcores
    n_workers = num_cores * num_subcores
    n_rows = BH * M
    rows_per_worker = n_rows // n_workers

    sc_topk = functools.partial(
        pl.kernel,
        mesh=plsc.VectorSubcoreMesh(core_axis_name="c", subcore_axis_name="s"),
        out_type=jax.ShapeDtypeStruct((n_rows, _LANES), jnp.int32),
        scratch_types=[pltpu.VMEM((rows_per_worker, M), jnp.float32),
                       pltpu.VMEM((rows_per_worker, _LANES), jnp.int32)],
    )(functools.partial(_sc_topk_body, rows_per_worker=rows_per_worker,
                        num_cores=num_cores))
    lut16 = sc_topk(scores.reshape(n_rows, M))

    lut_flat = lut16.reshape(n_rows * _LANES)

    o = pl.pallas_call(
        functools.partial(_attn_body, M=M, scale=1.0 / np.sqrt(D)),
        grid_spec=pltpu.PrefetchScalarGridSpec(
            num_scalar_prefetch=1,
            grid=(BH,),
            in_specs=[
                pl.BlockSpec((1, L, D), lambda bh, lut: (bh, 0, 0)),
                pl.BlockSpec((1, L, D), lambda bh, lut: (bh, 0, 0)),
                pl.BlockSpec((1, L, D), lambda bh, lut: (bh, 0, 0)),
            ],
            out_specs=pl.BlockSpec((1, L, D), lambda bh, lut: (bh, 0, 0)),
            scratch_shapes=[pltpu.VMEM((L, D), jnp.bfloat16),
                            pltpu.VMEM((L, D), jnp.bfloat16),
                            pltpu.VMEM((L, _TOPK * _BLOCK), jnp.float32),
                            pltpu.VMEM((L, _TOPK * _BLOCK), jnp.bfloat16)],
        ),
        out_shape=jax.ShapeDtypeStruct((BH, L, D), jnp.float32),
    )(lut_flat, q3, k3, v3)

    return o.reshape(B, H, L, D)


# speedup vs baseline: 1.0187x; 1.0187x over previous
import functools

import jax
import jax.numpy as jnp
import numpy as np
from jax import lax
from jax.experimental import pallas as pl
from jax.experimental.pallas import tpu as pltpu
from jax.experimental.pallas import tpu_sc as plsc

_BLOCK = 64
_TOPK = 8
_LANES = 16


def _scores_body(q_ref, k_ref, s_ref, *, M, L, D, G):
    for g in range(G):
        qp = jnp.mean(q_ref[g].reshape(M, _BLOCK, D), axis=1)
        kp = jnp.mean(k_ref[g].reshape(M, _BLOCK, D), axis=1)
        s_ref[g] = jax.lax.dot_general(
            qp.astype(jnp.bfloat16), kp.astype(jnp.bfloat16),
            (((1,), (1,)), ((), ())),
            preferred_element_type=jnp.float32)


def _sc_topk_body(scores_hbm, lut_hbm, sv_ref, lv_ref, *, rows_per_worker,
                  num_cores):
    wid = lax.axis_index("s") * num_cores + lax.axis_index("c")
    base = wid * rows_per_worker
    pltpu.sync_copy(scores_hbm.at[pl.ds(base, rows_per_worker)], sv_ref)
    ii = lax.iota(jnp.int32, _LANES)
    neg_inf = jnp.float32(-jnp.inf)

    gdims = lax.GatherDimensionNumbers(
        offset_dims=(), collapsed_slice_dims=(0,), start_index_map=(0,))

    def _rot(x, sh):
        perm = (ii + sh) & (_LANES - 1)
        return lax.gather(x, perm[:, None], gdims, slice_sizes=(1,),
                          mode=lax.GatherScatterMode.PROMISE_IN_BOUNDS)

    def vmax_splat(x):
        for sh in (8, 4, 2, 1):
            x = jnp.maximum(x, _rot(x, sh))
        return x

    def vmin_splat(x):
        for sh in (8, 4, 2, 1):
            x = jnp.minimum(x, _rot(x, sh))
        return x

    def first_idx_of(x, mx):
        return vmin_splat(jnp.where(x == mx, ii, 2 * _LANES))

    for r in range(rows_per_worker):
        a = sv_ref[r, pl.ds(0, _LANES)]
        b = sv_ref[r, pl.ds(_LANES, _LANES)]
        lutrow = jnp.zeros((_LANES,), jnp.int32)
        for t in range(_TOPK):
            gm = vmax_splat(jnp.maximum(a, b))
            fa = first_idx_of(a, gm)
            fb = first_idx_of(b, gm)
            idxv = jnp.minimum(fa, fb + _LANES)
            a = jnp.where(ii == idxv, neg_inf, a)
            b = jnp.where(ii + _LANES == idxv, neg_inf, b)
            lutrow = jnp.where(ii == t, idxv, lutrow)
        lv_ref[r, :] = lutrow
    pltpu.sync_copy(lv_ref, lut_hbm.at[pl.ds(base, rows_per_worker)])


def _attn_body(lut_ref, q_ref, k_ref, v_ref, o_ref, kb_ref, vb_ref,
               s_ref, p_ref, *, M, scale):
    bh = pl.program_id(0)
    kb_ref[...] = k_ref[0].astype(jnp.bfloat16)
    vb_ref[...] = v_ref[0].astype(jnp.bfloat16)
    for m in range(M):
        base = (bh * M + m) * _LANES
        q = q_ref[0, m * _BLOCK:(m + 1) * _BLOCK, :].astype(jnp.bfloat16)
        kcat = jnp.concatenate(
            [kb_ref[pl.ds(lut_ref[base + t] * _BLOCK, _BLOCK), :]
             for t in range(_TOPK)], axis=0)
        s_ref[m * _BLOCK:(m + 1) * _BLOCK, :] = jax.lax.dot_general(
            q, kcat, (((1,), (1,)), ((), ())),
            preferred_element_type=jnp.float32) * scale
    for m in range(M):
        s = s_ref[m * _BLOCK:(m + 1) * _BLOCK, :]
        mx = jnp.max(s, axis=-1, keepdims=True)
        p = jnp.exp(s - mx)
        se = jnp.sum(p, axis=-1, keepdims=True)
        p_ref[m * _BLOCK:(m + 1) * _BLOCK, :] = (p / se).astype(jnp.bfloat16)
    for m in range(M):
        base = (bh * M + m) * _LANES
        vcat = jnp.concatenate(
            [vb_ref[pl.ds(lut_ref[base + t] * _BLOCK, _BLOCK), :]
             for t in range(_TOPK)], axis=0)
        o_ref[0, m * _BLOCK:(m + 1) * _BLOCK, :] = jax.lax.dot_general(
            p_ref[m * _BLOCK:(m + 1) * _BLOCK, :], vcat,
            (((1,), (0,)), ((), ())), preferred_element_type=jnp.float32)


def kernel(q, k, v):
    B, H, L, D = q.shape
    M = L // _BLOCK
    BH = B * H
    q3 = q.reshape(BH, L, D)
    k3 = k.reshape(BH, L, D)
    v3 = v.reshape(BH, L, D)

    info = plsc.get_sparse_core_info()
    num_cores, num_subcores = info.num_cores, info.num_subcores
    n_workers = num_cores * num_subcores

    G = 4
    CH = 2
    chunk = BH // CH
    c_rows = chunk * M
    rows_per_worker = c_rows // n_workers

    sc_topk = functools.partial(
        pl.kernel,
        mesh=plsc.VectorSubcoreMesh(core_axis_name="c", subcore_axis_name="s"),
        out_type=jax.ShapeDtypeStruct((c_rows, _LANES), jnp.int32),
        scratch_types=[pltpu.VMEM((rows_per_worker, M), jnp.float32),
                       pltpu.VMEM((rows_per_worker, _LANES), jnp.int32)],
    )(functools.partial(_sc_topk_body, rows_per_worker=rows_per_worker,
                        num_cores=num_cores))

    lut_parts = []
    for c in range(CH):
        off = c * (chunk // G)
        scores_c = pl.pallas_call(
            functools.partial(_scores_body, M=M, L=L, D=D, G=G),
            grid=(chunk // G,),
            in_specs=[pl.BlockSpec((G, L, D), lambda i, off=off: (i + off, 0, 0)),
                      pl.BlockSpec((G, L, D), lambda i, off=off: (i + off, 0, 0))],
            out_specs=pl.BlockSpec((G, M, M), lambda i: (i, 0, 0)),
            out_shape=jax.ShapeDtypeStruct((chunk, M, M), jnp.float32),
        )(q3, k3)
        lut_parts.append(sc_topk(scores_c.reshape(c_rows, M)))

    lut_flat = jnp.concatenate(lut_parts, axis=0).reshape(BH * M * _LANES)

    o = pl.pallas_call(
        functools.partial(_attn_body, M=M, scale=1.0 / np.sqrt(D)),
        grid_spec=pltpu.PrefetchScalarGridSpec(
            num_scalar_prefetch=1,
            grid=(BH,),
            in_specs=[
                pl.BlockSpec((1, L, D), lambda bh, lut: (bh, 0, 0)),
                pl.BlockSpec((1, L, D), lambda bh, lut: (bh, 0, 0)),
                pl.BlockSpec((1, L, D), lambda bh, lut: (bh, 0, 0)),
            ],
            out_specs=pl.BlockSpec((1, L, D), lambda bh, lut: (bh, 0, 0)),
            scratch_shapes=[pltpu.VMEM((L, D), jnp.bfloat16),
                            pltpu.VMEM((L, D), jnp.bfloat16),
                            pltpu.VMEM((L, _TOPK * _BLOCK), jnp.float32),
                            pltpu.VMEM((L, _TOPK * _BLOCK), jnp.bfloat16)],
        ),
        out_shape=jax.ShapeDtypeStruct((BH, L, D), jnp.float32),
    )(lut_flat, q3, k3, v3)

    return o.reshape(B, H, L, D)


# speedup vs baseline: 1.0387x; 1.0196x over previous
import functools

import jax
import jax.numpy as jnp
import numpy as np
from jax import lax
from jax.experimental import pallas as pl
from jax.experimental.pallas import tpu as pltpu
from jax.experimental.pallas import tpu_sc as plsc

_BLOCK = 64
_TOPK = 8
_LANES = 16


def _scores_body(q_ref, k_ref, s_ref, *, M, L, D, G):
    for g in range(G):
        qp = jnp.mean(q_ref[g].reshape(M, _BLOCK, D), axis=1)
        kp = jnp.mean(k_ref[g].reshape(M, _BLOCK, D), axis=1)
        s_ref[g] = jax.lax.dot_general(
            qp.astype(jnp.bfloat16), kp.astype(jnp.bfloat16),
            (((1,), (1,)), ((), ())),
            preferred_element_type=jnp.float32)


def _sc_topk_body(scores_hbm, lut_hbm, sv_ref, lv_ref, *, rows_per_worker,
                  num_cores):
    wid = lax.axis_index("s") * num_cores + lax.axis_index("c")
    base = wid * rows_per_worker
    pltpu.sync_copy(scores_hbm.at[pl.ds(base, rows_per_worker)], sv_ref)
    ii = lax.iota(jnp.int32, _LANES)
    neg_inf = jnp.float32(-jnp.inf)

    gdims = lax.GatherDimensionNumbers(
        offset_dims=(), collapsed_slice_dims=(0,), start_index_map=(0,))

    def _rot(x, sh):
        perm = (ii + sh) & (_LANES - 1)
        return lax.gather(x, perm[:, None], gdims, slice_sizes=(1,),
                          mode=lax.GatherScatterMode.PROMISE_IN_BOUNDS)

    def vmax_splat(x):
        for sh in (8, 4, 2, 1):
            x = jnp.maximum(x, _rot(x, sh))
        return x

    def vmin_splat(x):
        for sh in (8, 4, 2, 1):
            x = jnp.minimum(x, _rot(x, sh))
        return x

    def first_idx_of(x, mx):
        return vmin_splat(jnp.where(x == mx, ii, 2 * _LANES))

    for r in range(rows_per_worker):
        a = sv_ref[r, pl.ds(0, _LANES)]
        b = sv_ref[r, pl.ds(_LANES, _LANES)]
        lutrow = jnp.zeros((_LANES,), jnp.int32)
        for t in range(_TOPK):
            gm = vmax_splat(jnp.maximum(a, b))
            fa = first_idx_of(a, gm)
            fb = first_idx_of(b, gm)
            idxv = jnp.minimum(fa, fb + _LANES)
            a = jnp.where(ii == idxv, neg_inf, a)
            b = jnp.where(ii + _LANES == idxv, neg_inf, b)
            lutrow = jnp.where(ii == t, idxv, lutrow)
        lv_ref[r, :] = lutrow
    pltpu.sync_copy(lv_ref, lut_hbm.at[pl.ds(base, rows_per_worker)])


def _attn_body(lut_ref, q_ref, k_ref, v_ref, *rest, M, scale, has_prev):
    if has_prev:
        _prev_ref, o_ref, kb_ref, vb_ref, s_ref, p_ref = rest
    else:
        o_ref, kb_ref, vb_ref, s_ref, p_ref = rest
    bh = pl.program_id(0)
    kb_ref[...] = k_ref[0].astype(jnp.bfloat16)
    vb_ref[...] = v_ref[0].astype(jnp.bfloat16)
    for m in range(M):
        base = (bh * M + m) * _LANES
        q = q_ref[0, m * _BLOCK:(m + 1) * _BLOCK, :].astype(jnp.bfloat16)
        kcat = jnp.concatenate(
            [kb_ref[pl.ds(lut_ref[base + t] * _BLOCK, _BLOCK), :]
             for t in range(_TOPK)], axis=0)
        s_ref[m * _BLOCK:(m + 1) * _BLOCK, :] = jax.lax.dot_general(
            q, kcat, (((1,), (1,)), ((), ())),
            preferred_element_type=jnp.float32) * scale
    for m in range(M):
        s = s_ref[m * _BLOCK:(m + 1) * _BLOCK, :]
        mx = jnp.max(s, axis=-1, keepdims=True)
        p = jnp.exp(s - mx)
        se = jnp.sum(p, axis=-1, keepdims=True)
        p_ref[m * _BLOCK:(m + 1) * _BLOCK, :] = (p / se).astype(jnp.bfloat16)
    for m in range(M):
        base = (bh * M + m) * _LANES
        vcat = jnp.concatenate(
            [vb_ref[pl.ds(lut_ref[base + t] * _BLOCK, _BLOCK), :]
             for t in range(_TOPK)], axis=0)
        o_ref[0, m * _BLOCK:(m + 1) * _BLOCK, :] = jax.lax.dot_general(
            p_ref[m * _BLOCK:(m + 1) * _BLOCK, :], vcat,
            (((1,), (0,)), ((), ())), preferred_element_type=jnp.float32)


def kernel(q, k, v):
    B, H, L, D = q.shape
    M = L // _BLOCK
    BH = B * H
    q3 = q.reshape(BH, L, D)
    k3 = k.reshape(BH, L, D)
    v3 = v.reshape(BH, L, D)

    info = plsc.get_sparse_core_info()
    num_cores, num_subcores = info.num_cores, info.num_subcores
    n_workers = num_cores * num_subcores

    G = 4
    CH = 2
    chunk = BH // CH
    c_rows = chunk * M
    rows_per_worker = c_rows // n_workers

    sc_topk = functools.partial(
        pl.kernel,
        mesh=plsc.VectorSubcoreMesh(core_axis_name="c", subcore_axis_name="s"),
        out_type=jax.ShapeDtypeStruct((c_rows, _LANES), jnp.int32),
        scratch_types=[pltpu.VMEM((rows_per_worker, M), jnp.float32),
                       pltpu.VMEM((rows_per_worker, _LANES), jnp.int32)],
    )(functools.partial(_sc_topk_body, rows_per_worker=rows_per_worker,
                        num_cores=num_cores))

    lut_parts = []
    for c in range(CH):
        off = c * (chunk // G)
        scores_c = pl.pallas_call(
            functools.partial(_scores_body, M=M, L=L, D=D, G=G),
            grid=(chunk // G,),
            in_specs=[pl.BlockSpec((G, L, D), lambda i, off=off: (i + off, 0, 0)),
                      pl.BlockSpec((G, L, D), lambda i, off=off: (i + off, 0, 0))],
            out_specs=pl.BlockSpec((G, M, M), lambda i: (i, 0, 0)),
            out_shape=jax.ShapeDtypeStruct((chunk, M, M), jnp.float32),
        )(q3, k3)
        lut_parts.append(sc_topk(scores_c.reshape(c_rows, M)))

    o = None
    for c in range(CH):
        off = c * chunk
        lutf = lut_parts[c].reshape(c_rows * _LANES)
        in_specs = [
            pl.BlockSpec((1, L, D), lambda bh, lut, off=off: (bh + off, 0, 0)),
            pl.BlockSpec((1, L, D), lambda bh, lut, off=off: (bh + off, 0, 0)),
            pl.BlockSpec((1, L, D), lambda bh, lut, off=off: (bh + off, 0, 0)),
        ]
        operands = [lutf, q3, k3, v3]
        kwargs = {}
        if o is not None:
            in_specs.append(pl.BlockSpec(memory_space=pl.ANY))
            operands.append(o)
            kwargs = dict(input_output_aliases={4: 0})
        o = pl.pallas_call(
            functools.partial(_attn_body, M=M, scale=1.0 / np.sqrt(D),
                              has_prev=o is not None),
            grid_spec=pltpu.PrefetchScalarGridSpec(
                num_scalar_prefetch=1,
                grid=(chunk,),
                in_specs=in_specs,
                out_specs=pl.BlockSpec((1, L, D),
                                       lambda bh, lut, off=off: (bh + off, 0, 0)),
                scratch_shapes=[pltpu.VMEM((L, D), jnp.bfloat16),
                                pltpu.VMEM((L, D), jnp.bfloat16),
                                pltpu.VMEM((L, _TOPK * _BLOCK), jnp.float32),
                                pltpu.VMEM((L, _TOPK * _BLOCK), jnp.bfloat16)],
            ),
            out_shape=jax.ShapeDtypeStruct((BH, L, D), jnp.float32),
            **kwargs,
        )(*operands)

    return o.reshape(B, H, L, D)
